# initial kernel scaffold (unmeasured)
import jax
import jax.numpy as jnp
from jax import lax
from jax.experimental import pallas as pl
from jax.experimental.pallas import tpu as pltpu

N_DEV = 4
H = 16
D = 128
SCALE = D ** -0.5


def kernel(Q, K, V):
    q = jnp.transpose(Q[0], (1, 0, 2))
    k = jnp.transpose(K[0], (1, 0, 2))
    v = jnp.transpose(V[0], (1, 0, 2))
    s_per = q.shape[1]

    def body(q_ref, k_ref, v_ref, out_ref, kbuf, vbuf, den,
             ksend, krecv, vsend, vrecv):
        my = lax.axis_index("i")
        left = (my - 1) % N_DEV
        right = (my + 1) % N_DEV

        barrier = pltpu.get_barrier_semaphore()
        for nbr in (left, right):
            pl.semaphore_signal(
                barrier, inc=1,
                device_id=(nbr,), device_id_type=pl.DeviceIdType.MESH,
            )
        pl.semaphore_wait(barrier, 2)

        def attend(chunk):
            def head_body(h, _):
                qh = q_ref[h]
                if chunk == 0:
                    kh, vh = k_ref[h], v_ref[h]
                else:
                    kh, vh = kbuf[chunk - 1, h], vbuf[chunk - 1, h]
                s = lax.dot_general(
                    qh, kh, (((1,), (1,)), ((), ())),
                    preferred_element_type=jnp.float32,
                ) * SCALE
                p = jnp.exp(s)
                pv = lax.dot_general(
                    p, vh, (((1,), (0,)), ((), ())),
                    preferred_element_type=jnp.float32,
                )
                ps = jnp.sum(p, axis=1, keepdims=True)
                if chunk == 0:
                    out_ref[h] = pv
                    den[h] = ps
                else:
                    out_ref[h] = out_ref[h] + pv
                    den[h] = den[h] + ps
                return 0
            lax.fori_loop(0, H, head_body, 0)

        for hop in range(N_DEV - 1):
            ksrc = k_ref if hop == 0 else kbuf.at[hop - 1]
            vsrc = v_ref if hop == 0 else vbuf.at[hop - 1]
            k_rdma = pltpu.make_async_remote_copy(
                src_ref=ksrc, dst_ref=kbuf.at[hop],
                send_sem=ksend.at[hop], recv_sem=krecv.at[hop],
                device_id=(right,), device_id_type=pl.DeviceIdType.MESH,
            )
            v_rdma = pltpu.make_async_remote_copy(
                src_ref=vsrc, dst_ref=vbuf.at[hop],
                send_sem=vsend.at[hop], recv_sem=vrecv.at[hop],
                device_id=(right,), device_id_type=pl.DeviceIdType.MESH,
            )
            k_rdma.start()
            v_rdma.start()
            attend(hop)
            k_rdma.wait()
            v_rdma.wait()
        attend(N_DEV - 1)

        def norm_body(h, _):
            out_ref[h] = out_ref[h] / den[h]
            return 0
        lax.fori_loop(0, H, norm_body, 0)

    out = pl.pallas_call(
        body,
        out_shape=jax.ShapeDtypeStruct((H, s_per, D), jnp.float32),
        in_specs=[pl.BlockSpec(memory_space=pltpu.VMEM)] * 3,
        out_specs=pl.BlockSpec(memory_space=pltpu.VMEM),
        scratch_shapes=[
            pltpu.VMEM((N_DEV - 1, H, s_per, D), jnp.float32),
            pltpu.VMEM((N_DEV - 1, H, s_per, D), jnp.float32),
            pltpu.VMEM((H, s_per, 1), jnp.float32),
            pltpu.SemaphoreType.DMA((N_DEV - 1,)),
            pltpu.SemaphoreType.DMA((N_DEV - 1,)),
            pltpu.SemaphoreType.DMA((N_DEV - 1,)),
            pltpu.SemaphoreType.DMA((N_DEV - 1,)),
        ],
        compiler_params=pltpu.CompilerParams(collective_id=0),
    )(q, k, v)
    return jnp.transpose(out, (1, 0, 2))[None]


# baseline (device time: 340717 ns/iter reference)
import jax
import jax.numpy as jnp
from jax import lax
from jax.experimental import pallas as pl
from jax.experimental.pallas import tpu as pltpu

N_DEV = 4
H = 16
D = 128
SCALE = D ** -0.5


def kernel(Q, K, V):
    q = jnp.transpose(Q[0], (1, 0, 2)).astype(jnp.bfloat16)
    k = jnp.transpose(K[0], (1, 0, 2)).astype(jnp.bfloat16)
    v = jnp.transpose(V[0], (1, 0, 2)).astype(jnp.bfloat16)
    s_per = q.shape[1]

    def body(q_ref, k_ref, v_ref, out_ref, kbuf, vbuf, den,
             ksend, krecv, vsend, vrecv):
        my = lax.axis_index("i")
        left = (my - 1) % N_DEV
        right = (my + 1) % N_DEV

        barrier = pltpu.get_barrier_semaphore()
        for nbr in (left, right):
            pl.semaphore_signal(
                barrier, inc=1,
                device_id=(nbr,), device_id_type=pl.DeviceIdType.MESH,
            )
        pl.semaphore_wait(barrier, 2)

        def attend(chunk):
            def head_body(h, _):
                qh = q_ref[h]
                if chunk == 0:
                    kh, vh = k_ref[h], v_ref[h]
                else:
                    kh, vh = kbuf[chunk - 1, h], vbuf[chunk - 1, h]
                s = lax.dot_general(
                    qh, kh, (((1,), (1,)), ((), ())),
                    preferred_element_type=jnp.float32,
                ) * SCALE
                p = jnp.exp(s)
                pb = p.astype(jnp.bfloat16)
                pv = lax.dot_general(
                    pb, vh, (((1,), (0,)), ((), ())),
                    preferred_element_type=jnp.float32,
                )
                ps = jnp.sum(p, axis=1)
                if chunk == 0:
                    out_ref[h] = pv
                    den[h] = ps
                else:
                    out_ref[h] = out_ref[h] + pv
                    den[h] = den[h] + ps
                return 0
            lax.fori_loop(0, H, head_body, 0)

        for hop in range(N_DEV - 1):
            ksrc = k_ref if hop == 0 else kbuf.at[hop - 1]
            vsrc = v_ref if hop == 0 else vbuf.at[hop - 1]
            k_rdma = pltpu.make_async_remote_copy(
                src_ref=ksrc, dst_ref=kbuf.at[hop],
                send_sem=ksend.at[hop], recv_sem=krecv.at[hop],
                device_id=(right,), device_id_type=pl.DeviceIdType.MESH,
            )
            v_rdma = pltpu.make_async_remote_copy(
                src_ref=vsrc, dst_ref=vbuf.at[hop],
                send_sem=vsend.at[hop], recv_sem=vrecv.at[hop],
                device_id=(right,), device_id_type=pl.DeviceIdType.MESH,
            )
            k_rdma.start()
            v_rdma.start()
            attend(hop)
            k_rdma.wait()
            v_rdma.wait()
        attend(N_DEV - 1)

        def norm_body(h, _):
            out_ref[h] = out_ref[h] / den[h][:, None]
            return 0
        lax.fori_loop(0, H, norm_body, 0)

    out = pl.pallas_call(
        body,
        out_shape=jax.ShapeDtypeStruct((H, s_per, D), jnp.float32),
        in_specs=[pl.BlockSpec(memory_space=pltpu.VMEM)] * 3,
        out_specs=pl.BlockSpec(memory_space=pltpu.VMEM),
        scratch_shapes=[
            pltpu.VMEM((N_DEV - 1, H, s_per, D), jnp.bfloat16),
            pltpu.VMEM((N_DEV - 1, H, s_per, D), jnp.bfloat16),
            pltpu.VMEM((H, s_per), jnp.float32),
            pltpu.SemaphoreType.DMA((N_DEV - 1,)),
            pltpu.SemaphoreType.DMA((N_DEV - 1,)),
            pltpu.SemaphoreType.DMA((N_DEV - 1,)),
            pltpu.SemaphoreType.DMA((N_DEV - 1,)),
        ],
        compiler_params=pltpu.CompilerParams(collective_id=0),
    )(q, k, v)
    return jnp.transpose(out, (1, 0, 2))[None]


# device time: 229968 ns/iter; 1.4816x vs baseline; 1.4816x over previous
import jax
import jax.numpy as jnp
from jax import lax
from jax.experimental import pallas as pl
from jax.experimental.pallas import tpu as pltpu

N_DEV = 4
H = 16
HH = H // 2
D = 128
SCALE = D ** -0.5


def kernel(Q, K, V):
    q = jnp.transpose(Q[0], (1, 0, 2)).astype(jnp.bfloat16)
    k = jnp.transpose(K[0], (1, 0, 2)).astype(jnp.bfloat16)
    v = jnp.transpose(V[0], (1, 0, 2)).astype(jnp.bfloat16)
    s_per = q.shape[1]

    def body(q_ref, k_ref, v_ref, out_ref, kbuf, vbuf, den,
             ksend, krecv, vsend, vrecv):
        my = lax.axis_index("i")
        left = (my - 1) % N_DEV
        right = (my + 1) % N_DEV

        barrier = pltpu.get_barrier_semaphore()
        for nbr in (left, right):
            pl.semaphore_signal(
                barrier, inc=1,
                device_id=(nbr,), device_id_type=pl.DeviceIdType.MESH,
            )
        pl.semaphore_wait(barrier, 2)

        def attend(chunk):
            def head_body(h, _):
                qh = q_ref[h]
                if chunk < 0:
                    kh, vh = k_ref[h], v_ref[h]
                else:
                    kh, vh = kbuf[chunk, h], vbuf[chunk, h]
                s = lax.dot_general(
                    qh, kh, (((1,), (1,)), ((), ())),
                    preferred_element_type=jnp.float32,
                ) * SCALE
                p = jnp.exp(s)
                pb = p.astype(jnp.bfloat16)
                pv = lax.dot_general(
                    pb, vh, (((1,), (0,)), ((), ())),
                    preferred_element_type=jnp.float32,
                )
                ps = jnp.sum(p, axis=1)
                if chunk < 0:
                    out_ref[h] = pv
                    den[h] = ps
                else:
                    out_ref[h] = out_ref[h] + pv
                    den[h] = den[h] + ps
                return 0
            lax.fori_loop(0, H, head_body, 0)

        def rdma(src, dst, sends, recvs, idx, target):
            return pltpu.make_async_remote_copy(
                src_ref=src, dst_ref=dst,
                send_sem=sends.at[idx], recv_sem=recvs.at[idx],
                device_id=(target,), device_id_type=pl.DeviceIdType.MESH,
            )

        p1 = [
            rdma(k_ref, kbuf.at[0], ksend, krecv, 0, right),
            rdma(v_ref, vbuf.at[0], vsend, vrecv, 0, right),
            rdma(k_ref, kbuf.at[1], ksend, krecv, 1, left),
            rdma(v_ref, vbuf.at[1], vsend, vrecv, 1, left),
        ]
        for r in p1:
            r.start()

        attend(-1)

        for r in p1:
            r.wait_recv()

        p2 = [
            rdma(kbuf.at[0, :HH], kbuf.at[2, :HH], ksend, krecv, 2, right),
            rdma(vbuf.at[0, :HH], vbuf.at[2, :HH], vsend, vrecv, 2, right),
            rdma(kbuf.at[1, HH:], kbuf.at[2, HH:], ksend, krecv, 3, left),
            rdma(vbuf.at[1, HH:], vbuf.at[2, HH:], vsend, vrecv, 3, left),
        ]
        for r in p2:
            r.start()

        attend(0)
        attend(1)

        for r in p2:
            r.wait_recv()
        attend(2)

        def norm_body(h, _):
            out_ref[h] = out_ref[h] / den[h][:, None]
            return 0
        lax.fori_loop(0, H, norm_body, 0)

        for r in p1 + p2:
            r.wait_send()

    out = pl.pallas_call(
        body,
        out_shape=jax.ShapeDtypeStruct((H, s_per, D), jnp.float32),
        in_specs=[pl.BlockSpec(memory_space=pltpu.VMEM)] * 3,
        out_specs=pl.BlockSpec(memory_space=pltpu.VMEM),
        scratch_shapes=[
            pltpu.VMEM((3, H, s_per, D), jnp.bfloat16),
            pltpu.VMEM((3, H, s_per, D), jnp.bfloat16),
            pltpu.VMEM((H, s_per), jnp.float32),
            pltpu.SemaphoreType.DMA((4,)),
            pltpu.SemaphoreType.DMA((4,)),
            pltpu.SemaphoreType.DMA((4,)),
            pltpu.SemaphoreType.DMA((4,)),
        ],
        compiler_params=pltpu.CompilerParams(collective_id=0),
    )(q, k, v)
    return jnp.transpose(out, (1, 0, 2))[None]


# device time: 203764 ns/iter; 1.6721x vs baseline; 1.1286x over previous
import jax
import jax.numpy as jnp
from jax import lax
from jax.experimental import pallas as pl
from jax.experimental.pallas import tpu as pltpu

N_DEV = 4
H = 16
HH = H // 2
D = 128
SCALE = D ** -0.5


def kernel(Q, K, V):
    q = jnp.transpose(Q[0], (1, 0, 2)).astype(jnp.bfloat16)
    k = jnp.transpose(K[0], (1, 0, 2)).astype(jnp.bfloat16)
    v = jnp.transpose(V[0], (1, 0, 2)).astype(jnp.bfloat16)
    s_per = q.shape[1]

    def body(q_ref, k_ref, v_ref, out_ref, kbuf, vbuf, den,
             ksend, krecv, vsend, vrecv):
        my = lax.axis_index("i")
        left = (my - 1) % N_DEV
        right = (my + 1) % N_DEV

        barrier = pltpu.get_barrier_semaphore()
        for nbr in (left, right):
            pl.semaphore_signal(
                barrier, inc=1,
                device_id=(nbr,), device_id_type=pl.DeviceIdType.MESH,
            )
        pl.semaphore_wait(barrier, 2)

        def attend(chunk, h0, h1):
            def head_body(h, _):
                qh = q_ref[h]
                if chunk < 0:
                    kh, vh = k_ref[h], v_ref[h]
                else:
                    kh, vh = kbuf[chunk, h], vbuf[chunk, h]
                s = lax.dot_general(
                    qh, kh, (((1,), (1,)), ((), ())),
                    preferred_element_type=jnp.float32,
                ) * SCALE
                p = jnp.exp(s)
                pb = p.astype(jnp.bfloat16)
                pv = lax.dot_general(
                    pb, vh, (((1,), (0,)), ((), ())),
                    preferred_element_type=jnp.float32,
                )
                ps = jnp.sum(p, axis=1)
                if chunk < 0:
                    out_ref[h] = pv
                    den[h] = ps
                else:
                    out_ref[h] = out_ref[h] + pv
                    den[h] = den[h] + ps
                return 0
            lax.fori_loop(h0, h1, head_body, 0)

        def rdma(bufs, src, dst, idx, target):
            sends, recvs = (ksend, krecv) if bufs == "k" else (vsend, vrecv)
            return pltpu.make_async_remote_copy(
                src_ref=src, dst_ref=dst,
                send_sem=sends.at[idx], recv_sem=recvs.at[idx],
                device_id=(target,), device_id_type=pl.DeviceIdType.MESH,
            )

        lo, hi = slice(0, HH), slice(HH, H)
        k0 = rdma("k", k_ref.at[lo], kbuf.at[0, lo], 0, right)
        v0 = rdma("v", v_ref.at[lo], vbuf.at[0, lo], 0, right)
        k1 = rdma("k", k_ref.at[hi], kbuf.at[0, hi], 1, right)
        v1 = rdma("v", v_ref.at[hi], vbuf.at[0, hi], 1, right)
        k2 = rdma("k", k_ref.at[lo], kbuf.at[1, lo], 2, left)
        v2 = rdma("v", v_ref.at[lo], vbuf.at[1, lo], 2, left)
        k3 = rdma("k", k_ref.at[hi], kbuf.at[1, hi], 3, left)
        v3 = rdma("v", v_ref.at[hi], vbuf.at[1, hi], 3, left)
        for r in (k0, v0, k2, v2, k1, v1, k3, v3):
            r.start()

        attend(-1, 0, H)

        k0.wait_recv(); v0.wait_recv()
        k4 = rdma("k", kbuf.at[0, lo], kbuf.at[2, lo], 4, right)
        v4 = rdma("v", vbuf.at[0, lo], vbuf.at[2, lo], 4, right)
        k4.start(); v4.start()
        attend(0, 0, HH)

        k2.wait_recv(); v2.wait_recv()
        attend(1, 0, HH)

        k1.wait_recv(); v1.wait_recv()
        attend(0, HH, H)

        k3.wait_recv(); v3.wait_recv()
        k5 = rdma("k", kbuf.at[1, hi], kbuf.at[2, hi], 5, left)
        v5 = rdma("v", vbuf.at[1, hi], vbuf.at[2, hi], 5, left)
        k5.start(); v5.start()
        attend(1, HH, H)

        k4.wait_recv(); v4.wait_recv()
        attend(2, 0, HH)
        k5.wait_recv(); v5.wait_recv()
        attend(2, HH, H)

        def norm_body(h, _):
            out_ref[h] = out_ref[h] / den[h][:, None]
            return 0
        lax.fori_loop(0, H, norm_body, 0)

        for r in (k0, v0, k1, v1, k2, v2, k3, v3, k4, v4, k5, v5):
            r.wait_send()

    out = pl.pallas_call(
        body,
        out_shape=jax.ShapeDtypeStruct((H, s_per, D), jnp.float32),
        in_specs=[pl.BlockSpec(memory_space=pltpu.VMEM)] * 3,
        out_specs=pl.BlockSpec(memory_space=pltpu.VMEM),
        scratch_shapes=[
            pltpu.VMEM((3, H, s_per, D), jnp.bfloat16),
            pltpu.VMEM((3, H, s_per, D), jnp.bfloat16),
            pltpu.VMEM((H, s_per), jnp.float32),
            pltpu.SemaphoreType.DMA((6,)),
            pltpu.SemaphoreType.DMA((6,)),
            pltpu.SemaphoreType.DMA((6,)),
            pltpu.SemaphoreType.DMA((6,)),
        ],
        compiler_params=pltpu.CompilerParams(collective_id=0),
    )(q, k, v)
    return jnp.transpose(out, (1, 0, 2))[None]


# device time: 184732 ns/iter; 1.8444x vs baseline; 1.1030x over previous
import jax
import jax.numpy as jnp
from jax import lax
from jax.experimental import pallas as pl
from jax.experimental.pallas import tpu as pltpu

N_DEV = 4
H = 16
NQ = 4
QH = H // NQ
D = 128
SCALE = D ** -0.5


def kernel(Q, K, V):
    q = jnp.transpose(Q[0], (1, 0, 2)).astype(jnp.bfloat16)
    k = jnp.transpose(K[0], (1, 0, 2)).astype(jnp.bfloat16)
    v = jnp.transpose(V[0], (1, 0, 2)).astype(jnp.bfloat16)
    s_per = q.shape[1]

    def body(q_ref, k_ref, v_ref, out_ref, kbuf, vbuf, den,
             ksend, krecv, vsend, vrecv):
        my = lax.axis_index("i")
        left = (my - 1) % N_DEV
        right = (my + 1) % N_DEV

        barrier = pltpu.get_barrier_semaphore()
        for nbr in (left, right):
            pl.semaphore_signal(
                barrier, inc=1,
                device_id=(nbr,), device_id_type=pl.DeviceIdType.MESH,
            )
        pl.semaphore_wait(barrier, 2)

        def attend(chunk, h0, h1):
            def head_body(h, _):
                qh = q_ref[h]
                if chunk < 0:
                    kh, vh = k_ref[h], v_ref[h]
                else:
                    kh, vh = kbuf[chunk, h], vbuf[chunk, h]
                s = lax.dot_general(
                    qh, kh, (((1,), (1,)), ((), ())),
                    preferred_element_type=jnp.float32,
                ) * SCALE
                p = jnp.exp(s)
                pb = p.astype(jnp.bfloat16)
                pv = lax.dot_general(
                    pb, vh, (((1,), (0,)), ((), ())),
                    preferred_element_type=jnp.float32,
                )
                ps = jnp.sum(p, axis=1)
                if chunk < 0:
                    out_ref[h] = pv
                    den[h] = ps
                else:
                    out_ref[h] = out_ref[h] + pv
                    den[h] = den[h] + ps
                return 0
            lax.fori_loop(h0, h1, head_body, 0)

        def rdma(tensor, src, dst, idx, target):
            sends, recvs = (ksend, krecv) if tensor == "k" else (vsend, vrecv)
            return pltpu.make_async_remote_copy(
                src_ref=src, dst_ref=dst,
                send_sem=sends.at[idx], recv_sem=recvs.at[idx],
                device_id=(target,), device_id_type=pl.DeviceIdType.MESH,
            )

        def qs(i):
            return slice(i * QH, (i + 1) * QH)

        p1R = []
        p1L = []
        for i in range(NQ):
            p1R.append((
                rdma("k", k_ref.at[qs(i)], kbuf.at[0, qs(i)], i, right),
                rdma("v", v_ref.at[qs(i)], vbuf.at[0, qs(i)], i, right),
            ))
            p1L.append((
                rdma("k", k_ref.at[qs(i)], kbuf.at[1, qs(i)], 4 + i, left),
                rdma("v", v_ref.at[qs(i)], vbuf.at[1, qs(i)], 4 + i, left),
            ))
        for i in range(NQ):
            for r in p1R[i] + p1L[i]:
                r.start()

        attend(-1, 0, H)

        p2 = []
        for i in range(NQ):
            for r in p1R[i]:
                r.wait_recv()
            if i < 2:
                fk = rdma("k", kbuf.at[0, qs(i)], kbuf.at[2, qs(i)],
                          8 + i, right)
                fv = rdma("v", vbuf.at[0, qs(i)], vbuf.at[2, qs(i)],
                          8 + i, right)
                fk.start()
                fv.start()
                p2.append((fk, fv))
            attend(0, i * QH, (i + 1) * QH)
            for r in p1L[i]:
                r.wait_recv()
            if i >= 2:
                fk = rdma("k", kbuf.at[1, qs(i)], kbuf.at[2, qs(i)],
                          8 + i, left)
                fv = rdma("v", vbuf.at[1, qs(i)], vbuf.at[2, qs(i)],
                          8 + i, left)
                fk.start()
                fv.start()
                p2.append((fk, fv))
            attend(1, i * QH, (i + 1) * QH)

        for i in (0, 2, 1, 3):
            for r in p2[i]:
                r.wait_recv()
            attend(2, i * QH, (i + 1) * QH)

        def norm_body(h, _):
            out_ref[h] = out_ref[h] / den[h][:, None]
            return 0
        lax.fori_loop(0, H, norm_body, 0)

        for pair in p1R + p1L + p2:
            for r in pair:
                r.wait_send()

    out = pl.pallas_call(
        body,
        out_shape=jax.ShapeDtypeStruct((H, s_per, D), jnp.float32),
        in_specs=[pl.BlockSpec(memory_space=pltpu.VMEM)] * 3,
        out_specs=pl.BlockSpec(memory_space=pltpu.VMEM),
        scratch_shapes=[
            pltpu.VMEM((3, H, s_per, D), jnp.bfloat16),
            pltpu.VMEM((3, H, s_per, D), jnp.bfloat16),
            pltpu.VMEM((H, s_per), jnp.float32),
            pltpu.SemaphoreType.DMA((12,)),
            pltpu.SemaphoreType.DMA((12,)),
            pltpu.SemaphoreType.DMA((12,)),
            pltpu.SemaphoreType.DMA((12,)),
        ],
        compiler_params=pltpu.CompilerParams(collective_id=0),
    )(q, k, v)
    return jnp.transpose(out, (1, 0, 2))[None]


# device time: 183889 ns/iter; 1.8528x vs baseline; 1.0046x over previous
import jax
import jax.numpy as jnp
from jax import lax
from jax.experimental import pallas as pl
from jax.experimental.pallas import tpu as pltpu

N_DEV = 4
H = 16
NQ = 4
QH = H // NQ
EH = 2
D = 128
SCALE = D ** -0.5


def kernel(Q, K, V):
    q = (jnp.transpose(Q[0], (1, 0, 2)) * SCALE).astype(jnp.bfloat16)
    k = jnp.transpose(K[0], (1, 0, 2)).astype(jnp.bfloat16)
    v = jnp.transpose(V[0], (1, 0, 2)).astype(jnp.bfloat16)
    s_per = q.shape[1]

    def body(q_ref, k_ref, v_ref, out_ref, kbuf, vbuf, den,
             ksend, krecv, vsend, vrecv):
        my = lax.axis_index("i")
        left = (my - 1) % N_DEV
        right = (my + 1) % N_DEV

        barrier = pltpu.get_barrier_semaphore()
        for nbr in (left, right):
            pl.semaphore_signal(
                barrier, inc=1,
                device_id=(nbr,), device_id_type=pl.DeviceIdType.MESH,
            )
        pl.semaphore_wait(barrier, 2)

        ones_bf = jnp.ones((s_per, D), jnp.bfloat16)

        def attend(chunk, h0, h1):
            def head_body(h, _):
                qh = q_ref[h]
                if chunk < 0:
                    kh, vh = k_ref[h], v_ref[h]
                else:
                    kh, vh = kbuf[chunk, h], vbuf[chunk, h]
                s = lax.dot_general(
                    qh, kh, (((1,), (1,)), ((), ())),
                    preferred_element_type=jnp.float32,
                )
                pb = jnp.exp(s.astype(jnp.bfloat16))
                pv = lax.dot_general(
                    pb, vh, (((1,), (0,)), ((), ())),
                    preferred_element_type=jnp.float32,
                )
                psc = lax.dot_general(
                    pb, ones_bf, (((1,), (0,)), ((), ())),
                    preferred_element_type=jnp.float32,
                )
                ps = psc[:, 0]
                if chunk < 0:
                    out_ref[h] = pv
                    den[h] = ps
                else:
                    out_ref[h] = out_ref[h] + pv
                    den[h] = den[h] + ps
                return 0
            lax.fori_loop(h0, h1, head_body, 0)

        def rdma(tensor, src, dst, idx, target):
            sends, recvs = (ksend, krecv) if tensor == "k" else (vsend, vrecv)
            return pltpu.make_async_remote_copy(
                src_ref=src, dst_ref=dst,
                send_sem=sends.at[idx], recv_sem=recvs.at[idx],
                device_id=(target,), device_id_type=pl.DeviceIdType.MESH,
            )

        def qsl(i):
            return slice(i * QH, (i + 1) * QH)

        def esl(j):
            return slice(j * EH, (j + 1) * EH)

        p1R = []
        p1L = []
        for i in range(NQ):
            p1R.append((
                rdma("k", k_ref.at[qsl(i)], kbuf.at[0, qsl(i)], i, right),
                rdma("v", v_ref.at[qsl(i)], vbuf.at[0, qsl(i)], i, right),
            ))
            p1L.append((
                rdma("k", k_ref.at[qsl(i)], kbuf.at[1, qsl(i)], 4 + i, left),
                rdma("v", v_ref.at[qsl(i)], vbuf.at[1, qsl(i)], 4 + i, left),
            ))
        for i in range(NQ):
            for r in p1R[i] + p1L[i]:
                r.start()

        attend(-1, 0, H)

        p2 = {}
        for i in range(NQ):
            for r in p1R[i]:
                r.wait_recv()
            if i < 2:
                for j in (2 * i, 2 * i + 1):
                    fk = rdma("k", kbuf.at[0, esl(j)], kbuf.at[2, esl(j)],
                              8 + j, right)
                    fv = rdma("v", vbuf.at[0, esl(j)], vbuf.at[2, esl(j)],
                              8 + j, right)
                    fk.start()
                    fv.start()
                    p2[j] = (fk, fv)
            attend(0, i * QH, (i + 1) * QH)
            for r in p1L[i]:
                r.wait_recv()
            if i >= 2:
                for j in (2 * i, 2 * i + 1):
                    fk = rdma("k", kbuf.at[1, esl(j)], kbuf.at[2, esl(j)],
                              8 + j, left)
                    fv = rdma("v", vbuf.at[1, esl(j)], vbuf.at[2, esl(j)],
                              8 + j, left)
                    fk.start()
                    fv.start()
                    p2[j] = (fk, fv)
            attend(1, i * QH, (i + 1) * QH)

        for j in (0, 4, 1, 5, 2, 6, 3, 7):
            for r in p2[j]:
                r.wait_recv()
            attend(2, j * EH, (j + 1) * EH)

        def norm_body(h, _):
            out_ref[h] = out_ref[h] / den[h][:, None]
            return 0
        lax.fori_loop(0, H, norm_body, 0)

        for pair in p1R + p1L + list(p2.values()):
            for r in pair:
                r.wait_send()

    out = pl.pallas_call(
        body,
        out_shape=jax.ShapeDtypeStruct((H, s_per, D), jnp.float32),
        in_specs=[pl.BlockSpec(memory_space=pltpu.VMEM)] * 3,
        out_specs=pl.BlockSpec(memory_space=pltpu.VMEM),
        scratch_shapes=[
            pltpu.VMEM((3, H, s_per, D), jnp.bfloat16),
            pltpu.VMEM((3, H, s_per, D), jnp.bfloat16),
            pltpu.VMEM((H, s_per), jnp.float32),
            pltpu.SemaphoreType.DMA((16,)),
            pltpu.SemaphoreType.DMA((16,)),
            pltpu.SemaphoreType.DMA((16,)),
            pltpu.SemaphoreType.DMA((16,)),
        ],
        compiler_params=pltpu.CompilerParams(collective_id=0),
    )(q, k, v)
    return jnp.transpose(out, (1, 0, 2))[None]


# device time: 174469 ns/iter; 1.9529x vs baseline; 1.0540x over previous
import jax
import jax.numpy as jnp
from jax import lax
from jax.experimental import pallas as pl
from jax.experimental.pallas import tpu as pltpu

N_DEV = 4
H = 16
NQ = 4
QH = H // NQ
EH = 2
D = 128
SCALE = D ** -0.5


def kernel(Q, K, V):
    q = (jnp.transpose(Q[0], (1, 0, 2)) * SCALE).astype(jnp.bfloat16)
    k = jnp.transpose(K[0], (1, 0, 2)).astype(jnp.bfloat16)
    v = jnp.transpose(V[0], (1, 0, 2)).astype(jnp.bfloat16)
    s_per = q.shape[1]

    def body(q_ref, k_ref, v_ref, out_ref, kbuf, vbuf, den,
             ksend, krecv, vsend, vrecv):
        my = lax.axis_index("i")
        left = (my - 1) % N_DEV
        right = (my + 1) % N_DEV

        barrier = pltpu.get_barrier_semaphore()
        for nbr in (left, right):
            pl.semaphore_signal(
                barrier, inc=1,
                device_id=(nbr,), device_id_type=pl.DeviceIdType.MESH,
            )
        pl.semaphore_wait(barrier, 2)

        def attend(chunk, h0, h1):
            def head_body(h, _):
                qh = q_ref[h]
                if chunk < 0:
                    kh, vh = k_ref[h], v_ref[h]
                else:
                    kh, vh = kbuf[chunk, h], vbuf[chunk, h]
                s = lax.dot_general(
                    qh, kh, (((1,), (1,)), ((), ())),
                    preferred_element_type=jnp.float32,
                )
                pb = jnp.exp(s.astype(jnp.bfloat16))
                pv = lax.dot_general(
                    pb, vh, (((1,), (0,)), ((), ())),
                    preferred_element_type=jnp.float32,
                )
                ps = jnp.sum(pb, axis=1, dtype=jnp.float32)
                if chunk < 0:
                    out_ref[h] = pv
                    den[h] = ps
                else:
                    out_ref[h] = out_ref[h] + pv
                    den[h] = den[h] + ps
                return 0
            lax.fori_loop(h0, h1, head_body, 0, unroll=8)

        def rdma(tensor, src, dst, idx, target):
            sends, recvs = (ksend, krecv) if tensor == "k" else (vsend, vrecv)
            return pltpu.make_async_remote_copy(
                src_ref=src, dst_ref=dst,
                send_sem=sends.at[idx], recv_sem=recvs.at[idx],
                device_id=(target,), device_id_type=pl.DeviceIdType.MESH,
            )

        def qsl(i):
            return slice(i * QH, (i + 1) * QH)

        def esl(j):
            return slice(j * EH, (j + 1) * EH)

        p1R = []
        p1L = []
        for i in range(NQ):
            p1R.append((
                rdma("k", k_ref.at[qsl(i)], kbuf.at[0, qsl(i)], i, right),
                rdma("v", v_ref.at[qsl(i)], vbuf.at[0, qsl(i)], i, right),
            ))
            p1L.append((
                rdma("k", k_ref.at[qsl(i)], kbuf.at[1, qsl(i)], 4 + i, left),
                rdma("v", v_ref.at[qsl(i)], vbuf.at[1, qsl(i)], 4 + i, left),
            ))
        for i in range(NQ):
            for r in p1R[i] + p1L[i]:
                r.start()

        attend(-1, 0, H)

        p2 = {}
        for i in range(NQ):
            for r in p1R[i]:
                r.wait_recv()
            if i < 2:
                for j in (2 * i, 2 * i + 1):
                    fk = rdma("k", kbuf.at[0, esl(j)], kbuf.at[2, esl(j)],
                              8 + j, right)
                    fv = rdma("v", vbuf.at[0, esl(j)], vbuf.at[2, esl(j)],
                              8 + j, right)
                    fk.start()
                    fv.start()
                    p2[j] = (fk, fv)
            attend(0, i * QH, (i + 1) * QH)
            for r in p1L[i]:
                r.wait_recv()
            if i >= 2:
                for j in (2 * i, 2 * i + 1):
                    fk = rdma("k", kbuf.at[1, esl(j)], kbuf.at[2, esl(j)],
                              8 + j, left)
                    fv = rdma("v", vbuf.at[1, esl(j)], vbuf.at[2, esl(j)],
                              8 + j, left)
                    fk.start()
                    fv.start()
                    p2[j] = (fk, fv)
            attend(1, i * QH, (i + 1) * QH)

        for j in (0, 4, 1, 5, 2, 6, 3, 7):
            for r in p2[j]:
                r.wait_recv()
            attend(2, j * EH, (j + 1) * EH)

        def norm_body(h, _):
            out_ref[h] = out_ref[h] / den[h][:, None]
            return 0
        lax.fori_loop(0, H, norm_body, 0)

        for pair in p1R + p1L + list(p2.values()):
            for r in pair:
                r.wait_send()

    out = pl.pallas_call(
        body,
        out_shape=jax.ShapeDtypeStruct((H, s_per, D), jnp.float32),
        in_specs=[pl.BlockSpec(memory_space=pltpu.VMEM)] * 3,
        out_specs=pl.BlockSpec(memory_space=pltpu.VMEM),
        scratch_shapes=[
            pltpu.VMEM((3, H, s_per, D), jnp.bfloat16),
            pltpu.VMEM((3, H, s_per, D), jnp.bfloat16),
            pltpu.VMEM((H, s_per), jnp.float32),
            pltpu.SemaphoreType.DMA((16,)),
            pltpu.SemaphoreType.DMA((16,)),
            pltpu.SemaphoreType.DMA((16,)),
            pltpu.SemaphoreType.DMA((16,)),
        ],
        compiler_params=pltpu.CompilerParams(collective_id=0),
    )(q, k, v)
    return jnp.transpose(out, (1, 0, 2))[None]


# device time: 174462 ns/iter; 1.9530x vs baseline; 1.0000x over previous
import jax
import jax.numpy as jnp
from jax import lax
from jax.experimental import pallas as pl
from jax.experimental.pallas import tpu as pltpu

N_DEV = 4
H = 16
NQ = 4
QH = H // NQ
EH = 2
D = 128
SCALE = D ** -0.5


def kernel(Q, K, V):
    q = jnp.transpose((Q[0] * SCALE).astype(jnp.bfloat16), (1, 0, 2))
    k = jnp.transpose(K[0].astype(jnp.bfloat16), (1, 0, 2))
    v = jnp.transpose(V[0].astype(jnp.bfloat16), (1, 0, 2))
    s_per = q.shape[1]

    def body(q_ref, k_ref, v_ref, out_ref, kbuf, vbuf, den,
             ksend, krecv, vsend, vrecv):
        my = lax.axis_index("i")
        left = (my - 1) % N_DEV
        right = (my + 1) % N_DEV

        barrier = pltpu.get_barrier_semaphore()
        for nbr in (left, right):
            pl.semaphore_signal(
                barrier, inc=1,
                device_id=(nbr,), device_id_type=pl.DeviceIdType.MESH,
            )
        pl.semaphore_wait(barrier, 2)

        def attend(chunk, h0, h1):
            def head_body(h, _):
                qh = q_ref[h]
                if chunk < 0:
                    kh, vh = k_ref[h], v_ref[h]
                else:
                    kh, vh = kbuf[chunk, h], vbuf[chunk, h]
                s = lax.dot_general(
                    qh, kh, (((1,), (1,)), ((), ())),
                    preferred_element_type=jnp.float32,
                )
                pb = jnp.exp(s.astype(jnp.bfloat16))
                pv = lax.dot_general(
                    pb, vh, (((1,), (0,)), ((), ())),
                    preferred_element_type=jnp.float32,
                )
                ps = jnp.sum(pb, axis=1, dtype=jnp.float32)
                if chunk < 0:
                    out_ref[h] = pv
                    den[h] = ps
                else:
                    out_ref[h] = out_ref[h] + pv
                    den[h] = den[h] + ps
                return 0
            lax.fori_loop(h0, h1, head_body, 0, unroll=8)

        def rdma(tensor, src, dst, idx, target):
            sends, recvs = (ksend, krecv) if tensor == "k" else (vsend, vrecv)
            return pltpu.make_async_remote_copy(
                src_ref=src, dst_ref=dst,
                send_sem=sends.at[idx], recv_sem=recvs.at[idx],
                device_id=(target,), device_id_type=pl.DeviceIdType.MESH,
            )

        def qsl(i):
            return slice(i * QH, (i + 1) * QH)

        def esl(j):
            return slice(j * EH, (j + 1) * EH)

        p1R = []
        p1L = []
        for i in range(NQ):
            p1R.append((
                rdma("k", k_ref.at[qsl(i)], kbuf.at[0, qsl(i)], i, right),
                rdma("v", v_ref.at[qsl(i)], vbuf.at[0, qsl(i)], i, right),
            ))
            p1L.append((
                rdma("k", k_ref.at[qsl(i)], kbuf.at[1, qsl(i)], 4 + i, left),
                rdma("v", v_ref.at[qsl(i)], vbuf.at[1, qsl(i)], 4 + i, left),
            ))
        for i in range(NQ):
            for r in p1R[i] + p1L[i]:
                r.start()

        attend(-1, 0, H)

        p2 = {}
        for i in range(NQ):
            for r in p1R[i]:
                r.wait_recv()
            if i < 2:
                for j in (2 * i, 2 * i + 1):
                    fk = rdma("k", kbuf.at[0, esl(j)], kbuf.at[2, esl(j)],
                              8 + j, right)
                    fv = rdma("v", vbuf.at[0, esl(j)], vbuf.at[2, esl(j)],
                              8 + j, right)
                    fk.start()
                    fv.start()
                    p2[j] = (fk, fv)
            attend(0, i * QH, (i + 1) * QH)
            for r in p1L[i]:
                r.wait_recv()
            if i >= 2:
                for j in (2 * i, 2 * i + 1):
                    fk = rdma("k", kbuf.at[1, esl(j)], kbuf.at[2, esl(j)],
                              8 + j, left)
                    fv = rdma("v", vbuf.at[1, esl(j)], vbuf.at[2, esl(j)],
                              8 + j, left)
                    fk.start()
                    fv.start()
                    p2[j] = (fk, fv)
            attend(1, i * QH, (i + 1) * QH)

        for j in (0, 4, 1, 5, 2, 6, 3, 7):
            for r in p2[j]:
                r.wait_recv()
            attend(2, j * EH, (j + 1) * EH)

        def norm_body(h, _):
            out_ref[h] = out_ref[h] / den[h][:, None]
            return 0
        lax.fori_loop(0, H, norm_body, 0)

        for pair in p1R + p1L + list(p2.values()):
            for r in pair:
                r.wait_send()

    out = pl.pallas_call(
        body,
        out_shape=jax.ShapeDtypeStruct((H, s_per, D), jnp.float32),
        in_specs=[pl.BlockSpec(memory_space=pltpu.VMEM)] * 3,
        out_specs=pl.BlockSpec(memory_space=pltpu.VMEM),
        scratch_shapes=[
            pltpu.VMEM((3, H, s_per, D), jnp.bfloat16),
            pltpu.VMEM((3, H, s_per, D), jnp.bfloat16),
            pltpu.VMEM((H, s_per), jnp.float32),
            pltpu.SemaphoreType.DMA((16,)),
            pltpu.SemaphoreType.DMA((16,)),
            pltpu.SemaphoreType.DMA((16,)),
            pltpu.SemaphoreType.DMA((16,)),
        ],
        compiler_params=pltpu.CompilerParams(collective_id=0),
    )(q, k, v)
    return jnp.transpose(out, (1, 0, 2))[None]
